# Initial kernel scaffold; baseline (speedup 1.0000x reference)
import jax
import jax.numpy as jnp
from jax.experimental import pallas as pl


def kernel(points, edge_index, W1, a_src1, a_dst1, b1, W2, a_src2, a_dst2, b2, W3, a_src3, a_dst3, b3, W4, a_src4, a_dst4, b4, W5, a_src5, a_dst5, b5, W6, a_src6, a_dst6, b6):
    raise NotImplementedError("write your pallas kernel here")


# speedup vs baseline: 9.2303x; 9.2303x over previous
import functools

import jax
import jax.numpy as jnp
from jax import lax
from jax.experimental import pallas as pl
from jax.experimental.pallas import tpu as pltpu
from jax.experimental.pallas import tpu_sc as plsc

N = 10000
E = 320000
NCORE = 2
NSUB = 16
NW = NCORE * NSUB
EPT = E // NW
B = 80
NB = EPT // B
BN = 1000
NI = N // BN
CW = 128
KC = CW // 16
F32 = jnp.float32

_NCHA = {1: 1, 2: 3, 3: 9, 4: 3, 5: 1, 6: 1}
_DIMS = [128, 64, 256, 1024, 256, 64, 3]


def _chunk_cols(dout, cc):
    if cc == 0:
        return 0, min(dout, CW - 16), 16
    lo = (CW - 16) + (cc - 1) * CW
    return lo, min(dout, lo + CW), 0


def _make_tc(layer):
    ncha = _NCHA[layer]
    dout = _DIMS[layer]
    din = _DIMS[layer - 1]
    first = layer == 1
    if not first:
        dprev, nchap = din, _NCHA[layer - 1]

    def body(*refs):
        if first:
            x_ref, w_ref, as_ref, ad_ref, h3_ref, es_ref, ed_ref, h_s = refs
        else:
            (p_ref, b_ref, w_ref, as_ref, ad_ref,
             h3_ref, es_ref, ed_ref, h_s) = refs
        c = pl.program_id(1)

        @pl.when(c == 0)
        def _():
            if first:
                x = x_ref[...]
            else:
                pr = p_ref[...]
                p = pr[0] + pr[1]
                den = p[0, :, 0:1] + 1e-16
                cols = []
                rem = dprev
                for cc in range(nchap):
                    lo, hi, off = _chunk_cols(dprev, cc)
                    cols.append(p[cc, :, off:off + (hi - lo)])
                    rem -= hi - lo
                xa = cols[0] if len(cols) == 1 else jnp.concatenate(cols, 1)
                x = jnp.maximum(xa / den + b_ref[...], 0.0)
            h = jnp.dot(x, w_ref[...], preferred_element_type=F32)
            h_s[...] = h
            es_ref[...] = jnp.dot(h, as_ref[...], preferred_element_type=F32)
            ed_ref[...] = jnp.dot(h, ad_ref[...], preferred_element_type=F32)

        for cc in range(ncha):
            @pl.when(c == cc)
            def _():
                lo, hi, off = _chunk_cols(dout, cc)
                pieces = []
                if cc == 0:
                    pieces.append(jnp.ones((BN, 16), F32))
                pieces.append(h_s[:, lo:hi])
                pad = CW - off - (hi - lo)
                if pad:
                    pieces.append(jnp.zeros((BN, pad), F32))
                h3_ref[...] = jnp.concatenate(pieces, 1).reshape(1, BN, CW)

    w_spec = pl.BlockSpec((din, dout), lambda i, c: (0, 0))
    a_spec = pl.BlockSpec((dout, 1), lambda i, c: (0, 0))
    if first:
        in_specs = [pl.BlockSpec((BN, din), lambda i, c: (i, 0)),
                    w_spec, a_spec, a_spec]
    else:
        in_specs = [pl.BlockSpec((2, nchap, BN, CW), lambda i, c: (0, 0, i, 0)),
                    pl.BlockSpec((1, dprev), lambda i, c: (0, 0)),
                    w_spec, a_spec, a_spec]
    out_specs = [pl.BlockSpec((1, BN, CW), lambda i, c: (c, i, 0)),
                 pl.BlockSpec((BN, 1), lambda i, c: (i, 0)),
                 pl.BlockSpec((BN, 1), lambda i, c: (i, 0))]
    out_shape = [jax.ShapeDtypeStruct((ncha, N, CW), F32),
                 jax.ShapeDtypeStruct((N, 1), F32),
                 jax.ShapeDtypeStruct((N, 1), F32)]
    return pl.pallas_call(
        body, grid=(NI, ncha), in_specs=in_specs, out_specs=out_specs,
        out_shape=out_shape,
        scratch_shapes=[pltpu.VMEM((BN, dout), F32)],
        compiler_params=pltpu.CompilerParams(
            dimension_semantics=("arbitrary", "arbitrary")))


def _smax():
    def body(es_ref, s_ref):
        s_ref[...] = jnp.full((1, 16), jnp.max(es_ref[...]), F32)

    return pl.pallas_call(
        body,
        in_specs=[pl.BlockSpec((N, 1), lambda: (0, 0))],
        out_specs=pl.BlockSpec((1, 16), lambda: (0, 0)),
        out_shape=jax.ShapeDtypeStruct((1, 16), F32))


def _epilogue():
    def body(p_ref, b_ref, o_ref):
        pr = p_ref[...]
        p = pr[0, 0] + pr[1, 0]
        den = p[:, 0:1] + 1e-16
        o_ref[...] = p[:, 16:32] / den + b_ref[...]

    return pl.pallas_call(
        body, grid=(NI,),
        in_specs=[pl.BlockSpec((2, 1, BN, CW), lambda i: (0, 0, i, 0)),
                  pl.BlockSpec((1, 16), lambda i: (0, 0))],
        out_specs=pl.BlockSpec((BN, 16), lambda i: (i, 0)),
        out_shape=jax.ShapeDtypeStruct((N, 16), F32))


def _make_sc(layer):
    ncha = _NCHA[layer]
    mesh = plsc.VectorSubcoreMesh(core_axis_name="core",
                                  subcore_axis_name="subcore")

    @functools.partial(
        pl.kernel, mesh=mesh,
        compiler_params=pltpu.CompilerParams(needs_layout_passes=False),
        out_type=jax.ShapeDtypeStruct((NCORE, ncha, N, CW), F32),
        scratch_types=[
            pltpu.VMEM((N,), F32),
            pltpu.VMEM((N,), F32),
            pltpu.VMEM((16,), F32),
            pltpu.VMEM((1, B), jnp.int32),
            pltpu.VMEM((1, B), jnp.int32),
            pltpu.VMEM((B, CW), F32),
            pltpu.VMEM_SHARED((N, CW), F32),
        ])
    def k(h3_hbm, es_hbm, ed_hbm, smax_hbm, src_hbm, dst_hbm, out_hbm,
          es_v, ed_v, smax_v, srcb, dstb, rows_v, acc_sh):
        cid = lax.axis_index("core")
        sid = lax.axis_index("subcore")
        wid = cid * NSUB + sid
        pltpu.sync_copy(es_hbm, es_v)
        pltpu.sync_copy(ed_hbm, ed_v)
        pltpu.sync_copy(smax_hbm, smax_v)

        NU = N // 16
        for c in range(ncha):
            @pl.loop(0, 16)
            def _(r):
                for kk in range(KC):
                    rows_v[r, pl.ds(16 * kk, 16)] = jnp.zeros((16,), F32)

            @pl.loop(0, (NU + NSUB - 1) // NSUB)
            def _(k):
                u = sid + NSUB * k

                @pl.when(u < NU)
                def _():
                    off = pl.multiple_of(16 * u, 16)
                    pltpu.sync_copy(rows_v.at[pl.ds(0, 16)],
                                    acc_sh.at[pl.ds(off, 16)])

            plsc.subcore_barrier()

            @pl.loop(0, NB)
            def _(jb):
                pltpu.sync_copy(src_hbm.at[wid * NB + jb], srcb)
                pltpu.sync_copy(dst_hbm.at[wid * NB + jb], dstb)
                pltpu.sync_copy(h3_hbm.at[c].at[srcb.at[0]], rows_v)
                sv = smax_v[...]

                @pl.loop(0, B // 16)
                def _(q):
                    s16 = srcb[0, pl.ds(16 * q, 16)]
                    d16 = dstb[0, pl.ds(16 * q, 16)]
                    esg = plsc.load_gather(es_v, [s16])
                    edg = plsc.load_gather(ed_v, [d16])
                    uu = esg + edg
                    e = jnp.where(uu >= 0, uu, 0.2 * uu)
                    vv = edg + sv
                    mt = jnp.where(vv >= 0, vv, 0.2 * vv)
                    ex16 = jnp.exp(e - mt)
                    for t in range(16):
                        s = ex16[t]
                        for kk in range(KC):
                            rows_v[16 * q + t, pl.ds(16 * kk, 16)] = (
                                rows_v[16 * q + t, pl.ds(16 * kk, 16)] * s)

                pltpu.sync_copy(rows_v, acc_sh.at[dstb.at[0]], add=True)

            plsc.subcore_barrier()

            @pl.loop(0, (NU + NSUB - 1) // NSUB)
            def _(k):
                u = sid + NSUB * k

                @pl.when(u < NU)
                def _():
                    off = pl.multiple_of(16 * u, 16)
                    pltpu.sync_copy(acc_sh.at[pl.ds(off, 16)],
                                    out_hbm.at[cid, c, pl.ds(off, 16)])

    return k


_TC = {i: _make_tc(i) for i in range(1, 7)}
_SC = {i: _make_sc(i) for i in range(1, 7)}
_EPI = _epilogue()
_SMAX = _smax()


def kernel(points, edge_index, W1, a_src1, a_dst1, b1, W2, a_src2, a_dst2, b2,
           W3, a_src3, a_dst3, b3, W4, a_src4, a_dst4, b4,
           W5, a_src5, a_dst5, b5, W6, a_src6, a_dst6, b6):
    Ws = [W1, W2, W3, W4, W5, W6]
    ass = [a_src1, a_src2, a_src3, a_src4, a_src5, a_src6]
    ads = [a_dst1, a_dst2, a_dst3, a_dst4, a_dst5, a_dst6]
    bs = [b1, b2, b3, b4, b5, b6]
    x = points.reshape(N, _DIMS[0])
    srcr = edge_index[0].reshape(NW * NB, 1, B)
    dstr = edge_index[1].reshape(NW * NB, 1, B)
    parts = None
    for i in range(1, 7):
        W = Ws[i - 1]
        a_s = ass[i - 1].reshape(-1, 1)
        a_d = ads[i - 1].reshape(-1, 1)
        if i == 1:
            h3, es, ed = _TC[1](x, W, a_s, a_d)
        else:
            bprev = bs[i - 2].reshape(1, -1)
            h3, es, ed = _TC[i](parts, bprev, W, a_s, a_d)
        smax = _SMAX(es)
        parts = _SC[i](h3, es.reshape(N), ed.reshape(N), smax.reshape(16),
                       srcr, dstr)
    b6p = jnp.zeros((1, 16), F32).at[0, :3].set(b6)
    out16 = _EPI(parts, b6p)
    return out16[:, :3].reshape(1, N, 3)


# speedup vs baseline: 18.9410x; 2.0520x over previous
import functools

import jax
import jax.numpy as jnp
from jax import lax
from jax.experimental import pallas as pl
from jax.experimental.pallas import tpu as pltpu
from jax.experimental.pallas import tpu_sc as plsc

N = 10000
E = 320000
NCORE = 2
NSUB = 16
NW = NCORE * NSUB
EPT = E // NW
B = 80
NB = EPT // B
BN = 1000
NI = N // BN
CW = 128
KC = CW // 16
F32 = jnp.float32

_NCHA = {1: 1, 2: 3, 3: 9, 4: 3, 5: 1, 6: 1}
_DIMS = [128, 64, 256, 1024, 256, 64, 3]


def _chunk_cols(dout, cc):
    if cc == 0:
        return 0, min(dout, CW - 16), 16
    lo = (CW - 16) + (cc - 1) * CW
    return lo, min(dout, lo + CW), 0


def _make_tc(layer):
    ncha = _NCHA[layer]
    dout = _DIMS[layer]
    din = _DIMS[layer - 1]
    first = layer == 1
    if not first:
        dprev, nchap = din, _NCHA[layer - 1]

    def body(*refs):
        if first:
            x_ref, w_ref, as_ref, ad_ref, h3_ref, es_ref, ed_ref, h_s = refs
        else:
            (p_ref, b_ref, w_ref, as_ref, ad_ref,
             h3_ref, es_ref, ed_ref, h_s) = refs
        c = pl.program_id(1)

        @pl.when(c == 0)
        def _():
            if first:
                x = x_ref[...]
            else:
                pr = p_ref[...]
                p = pr[0] + pr[1]
                den = p[0, :, 0:1] + 1e-16
                cols = []
                rem = dprev
                for cc in range(nchap):
                    lo, hi, off = _chunk_cols(dprev, cc)
                    cols.append(p[cc, :, off:off + (hi - lo)])
                    rem -= hi - lo
                xa = cols[0] if len(cols) == 1 else jnp.concatenate(cols, 1)
                x = jnp.maximum(xa / den + b_ref[...], 0.0)
            h = jnp.dot(x, w_ref[...], preferred_element_type=F32)
            h_s[...] = h
            es_ref[...] = jnp.dot(h, as_ref[...], preferred_element_type=F32)
            ed_ref[...] = jnp.dot(h, ad_ref[...], preferred_element_type=F32)

        for cc in range(ncha):
            @pl.when(c == cc)
            def _():
                lo, hi, off = _chunk_cols(dout, cc)
                pieces = []
                if cc == 0:
                    pieces.append(jnp.ones((BN, 16), F32))
                pieces.append(h_s[:, lo:hi])
                pad = CW - off - (hi - lo)
                if pad:
                    pieces.append(jnp.zeros((BN, pad), F32))
                h3_ref[...] = jnp.concatenate(pieces, 1).reshape(1, BN, CW)

    w_spec = pl.BlockSpec((din, dout), lambda i, c: (0, 0))
    a_spec = pl.BlockSpec((dout, 1), lambda i, c: (0, 0))
    if first:
        in_specs = [pl.BlockSpec((BN, din), lambda i, c: (i, 0)),
                    w_spec, a_spec, a_spec]
    else:
        in_specs = [pl.BlockSpec((2, nchap, BN, CW), lambda i, c: (0, 0, i, 0)),
                    pl.BlockSpec((1, dprev), lambda i, c: (0, 0)),
                    w_spec, a_spec, a_spec]
    out_specs = [pl.BlockSpec((1, BN, CW), lambda i, c: (c, i, 0)),
                 pl.BlockSpec((BN, 1), lambda i, c: (i, 0)),
                 pl.BlockSpec((BN, 1), lambda i, c: (i, 0))]
    out_shape = [jax.ShapeDtypeStruct((ncha, N, CW), F32),
                 jax.ShapeDtypeStruct((N, 1), F32),
                 jax.ShapeDtypeStruct((N, 1), F32)]
    return pl.pallas_call(
        body, grid=(NI, ncha), in_specs=in_specs, out_specs=out_specs,
        out_shape=out_shape,
        scratch_shapes=[pltpu.VMEM((BN, dout), F32)],
        compiler_params=pltpu.CompilerParams(
            dimension_semantics=("arbitrary", "arbitrary")))


def _smax():
    def body(es_ref, s_ref):
        s_ref[...] = jnp.full((1, 16), jnp.max(es_ref[...]), F32)

    return pl.pallas_call(
        body,
        in_specs=[pl.BlockSpec((N, 1), lambda: (0, 0))],
        out_specs=pl.BlockSpec((1, 16), lambda: (0, 0)),
        out_shape=jax.ShapeDtypeStruct((1, 16), F32))


def _epilogue():
    def body(p_ref, b_ref, o_ref):
        pr = p_ref[...]
        p = pr[0, 0] + pr[1, 0]
        den = p[:, 0:1] + 1e-16
        o_ref[...] = p[:, 16:32] / den + b_ref[...]

    return pl.pallas_call(
        body, grid=(NI,),
        in_specs=[pl.BlockSpec((2, 1, BN, CW), lambda i: (0, 0, i, 0)),
                  pl.BlockSpec((1, 16), lambda i: (0, 0))],
        out_specs=pl.BlockSpec((BN, 16), lambda i: (i, 0)),
        out_shape=jax.ShapeDtypeStruct((N, 16), F32))


def _make_sc(layer):
    ncha = _NCHA[layer]
    dout = _DIMS[layer]
    mesh = plsc.VectorSubcoreMesh(core_axis_name="core",
                                  subcore_axis_name="subcore")
    kcs = []
    for cc in range(ncha):
        lo, hi, off = _chunk_cols(dout, cc)
        used = off + (hi - lo)
        kcs.append(-(-used // 16))

    @functools.partial(
        pl.kernel, mesh=mesh,
        compiler_params=pltpu.CompilerParams(needs_layout_passes=False),
        out_type=jax.ShapeDtypeStruct((NCORE, ncha, N, CW), F32),
        scratch_types=[
            pltpu.VMEM((N,), F32),
            pltpu.VMEM((N,), F32),
            pltpu.VMEM((16,), F32),
            pltpu.VMEM((2, B), jnp.int32),
            pltpu.VMEM((2, B), jnp.int32),
            pltpu.VMEM((B, CW), F32),
            pltpu.VMEM((B, CW), F32),
            pltpu.VMEM_SHARED((N, CW), F32),
            pltpu.SemaphoreType.DMA,
            pltpu.SemaphoreType.DMA,
            pltpu.SemaphoreType.DMA,
            pltpu.SemaphoreType.DMA,
            pltpu.SemaphoreType.DMA,
        ])
    def k(h3_hbm, es_hbm, ed_hbm, smax_hbm, sd_hbm, out_hbm,
          es_v, ed_v, smax_v, sd0, sd1, rows0, rows1, acc_sh,
          sem_g0, sem_g1, sem_s0, sem_s1, sem_b):
        cid = lax.axis_index("core")
        sid = lax.axis_index("subcore")
        wid = cid * NSUB + sid
        pltpu.sync_copy(es_hbm, es_v)
        pltpu.sync_copy(ed_hbm, ed_v)
        pltpu.sync_copy(smax_hbm, smax_v)
        sdb = (sd0, sd1)
        rows = (rows0, rows1)
        sem_g = (sem_g0, sem_g1)
        sem_s = (sem_s0, sem_s1)
        base = wid * NB

        NU = N // 16
        NK = (NU + NSUB - 1) // NSUB
        for c in range(ncha):
            kc = kcs[c]

            @pl.loop(0, 16)
            def _(r):
                for kk in range(KC):
                    rows0[r, pl.ds(16 * kk, 16)] = jnp.zeros((16,), F32)

            @pl.loop(0, NK)
            def _(k):
                u = sid + NSUB * k

                @pl.when(u < NU)
                def _():
                    off = pl.multiple_of(16 * u, 16)
                    pltpu.async_copy(rows0.at[pl.ds(0, 16)],
                                     acc_sh.at[pl.ds(off, 16)], sem_b)

            @pl.loop(0, NK)
            def _(k):
                u = sid + NSUB * k

                @pl.when(u < NU)
                def _():
                    off = pl.multiple_of(16 * u, 16)
                    pltpu.make_async_copy(rows0.at[pl.ds(0, 16)],
                                          acc_sh.at[pl.ds(off, 16)],
                                          sem_b).wait()

            plsc.subcore_barrier()

            pltpu.sync_copy(sd_hbm.at[base], sd0)
            pltpu.async_copy(h3_hbm.at[c].at[sd0.at[0]], rows0, sem_g0)

            @pl.loop(0, (NB + 1) // 2)
            def _(i):
                for bb in range(2):
                    jb = 2 * i + bb
                    p, q = bb, 1 - bb

                    @pl.when(jb < NB)
                    def _():
                        @pl.when(jb + 1 < NB)
                        def _():
                            @pl.when(jb >= 1)
                            def _():
                                pltpu.make_async_copy(
                                    rows[q], acc_sh.at[sdb[q].at[1]],
                                    sem_s[q]).wait()
                            pltpu.sync_copy(sd_hbm.at[base + jb + 1], sdb[q])
                            pltpu.async_copy(h3_hbm.at[c].at[sdb[q].at[0]],
                                             rows[q], sem_g[q])

                        pltpu.make_async_copy(h3_hbm.at[c].at[sdb[p].at[0]],
                                              rows[p], sem_g[p]).wait()
                        sv = smax_v[...]

                        @pl.loop(0, B // 16)
                        def _(qq):
                            s16 = sdb[p][0, pl.ds(16 * qq, 16)]
                            d16 = sdb[p][1, pl.ds(16 * qq, 16)]
                            esg = plsc.load_gather(es_v, [s16])
                            edg = plsc.load_gather(ed_v, [d16])
                            uu = esg + edg
                            e = jnp.where(uu >= 0, uu, 0.2 * uu)
                            vv = edg + sv
                            mt = jnp.where(vv >= 0, vv, 0.2 * vv)
                            ex16 = jnp.exp(e - mt)
                            for t in range(16):
                                s = ex16[t]
                                for kk in range(kc):
                                    rows[p][16 * qq + t, pl.ds(16 * kk, 16)] = (
                                        rows[p][16 * qq + t,
                                                pl.ds(16 * kk, 16)] * s)

                        pltpu.async_copy(rows[p], acc_sh.at[sdb[p].at[1]],
                                         sem_s[p], add=True)

            pltpu.make_async_copy(rows[0], acc_sh.at[sdb[0].at[1]],
                                  sem_s[0]).wait()
            pltpu.make_async_copy(rows[1], acc_sh.at[sdb[1].at[1]],
                                  sem_s[1]).wait()
            plsc.subcore_barrier()

            @pl.loop(0, NK)
            def _(k):
                u = sid + NSUB * k

                @pl.when(u < NU)
                def _():
                    off = pl.multiple_of(16 * u, 16)
                    pltpu.async_copy(acc_sh.at[pl.ds(off, 16)],
                                     out_hbm.at[cid, c, pl.ds(off, 16)],
                                     sem_b)

            @pl.loop(0, NK)
            def _(k):
                u = sid + NSUB * k

                @pl.when(u < NU)
                def _():
                    off = pl.multiple_of(16 * u, 16)
                    pltpu.make_async_copy(acc_sh.at[pl.ds(off, 16)],
                                          out_hbm.at[cid, c, pl.ds(off, 16)],
                                          sem_b).wait()

    return k


_TC = {i: _make_tc(i) for i in range(1, 7)}
_SC = {i: _make_sc(i) for i in range(1, 7)}
_EPI = _epilogue()
_SMAX = _smax()


def kernel(points, edge_index, W1, a_src1, a_dst1, b1, W2, a_src2, a_dst2, b2,
           W3, a_src3, a_dst3, b3, W4, a_src4, a_dst4, b4,
           W5, a_src5, a_dst5, b5, W6, a_src6, a_dst6, b6):
    Ws = [W1, W2, W3, W4, W5, W6]
    ass = [a_src1, a_src2, a_src3, a_src4, a_src5, a_src6]
    ads = [a_dst1, a_dst2, a_dst3, a_dst4, a_dst5, a_dst6]
    bs = [b1, b2, b3, b4, b5, b6]
    x = points.reshape(N, _DIMS[0])
    sdr = jnp.concatenate([edge_index[0].reshape(NW * NB, 1, B),
                           edge_index[1].reshape(NW * NB, 1, B)], 1)
    parts = None
    for i in range(1, 7):
        W = Ws[i - 1]
        a_s = ass[i - 1].reshape(-1, 1)
        a_d = ads[i - 1].reshape(-1, 1)
        if i == 1:
            h3, es, ed = _TC[1](x, W, a_s, a_d)
        else:
            bprev = bs[i - 2].reshape(1, -1)
            h3, es, ed = _TC[i](parts, bprev, W, a_s, a_d)
        smax = _SMAX(es)
        parts = _SC[i](h3, es.reshape(N), ed.reshape(N), smax.reshape(16),
                       sdr)
    b6p = jnp.zeros((1, 16), F32).at[0, :3].set(b6)
    out16 = _EPI(parts, b6p)
    return out16[:, :3].reshape(1, N, 3)


# speedup vs baseline: 24.1505x; 1.2750x over previous
import functools

import jax
import jax.numpy as jnp
from jax import lax
from jax.experimental import pallas as pl
from jax.experimental.pallas import tpu as pltpu
from jax.experimental.pallas import tpu_sc as plsc

N = 10000
E = 320000
NCORE = 2
NSUB = 16
NW = NCORE * NSUB
EPT = E // NW
B = 80
NB = EPT // B
BN = 1000
NI = N // BN
CW = 128
KC = CW // 16
F32 = jnp.float32

_NCHA = {1: 1, 2: 3, 3: 9, 4: 3, 5: 1, 6: 1}
_DIMS = [128, 64, 256, 1024, 256, 64, 3]


def _chunk_cols(dout, cc):
    if cc == 0:
        return 0, min(dout, CW - 16), 16
    lo = (CW - 16) + (cc - 1) * CW
    return lo, min(dout, lo + CW), 0


def _make_tc(layer):
    ncha = _NCHA[layer]
    dout = _DIMS[layer]
    din = _DIMS[layer - 1]
    first = layer == 1
    if not first:
        dprev, nchap = din, _NCHA[layer - 1]

    def body(*refs):
        if first:
            x_ref, w_ref, as_ref, ad_ref, h3_ref, es_ref, ed_ref, h_s = refs
        else:
            (p_ref, b_ref, w_ref, as_ref, ad_ref,
             h3_ref, es_ref, ed_ref, h_s) = refs
        c = pl.program_id(1)

        @pl.when(c == 0)
        def _():
            if first:
                x = x_ref[...]
            else:
                pr = p_ref[...]
                p = pr[0] + pr[1]
                den = p[0, :, 0:1] + 1e-16
                cols = []
                rem = dprev
                for cc in range(nchap):
                    lo, hi, off = _chunk_cols(dprev, cc)
                    cols.append(p[cc, :, off:off + (hi - lo)])
                    rem -= hi - lo
                xa = cols[0] if len(cols) == 1 else jnp.concatenate(cols, 1)
                x = jnp.maximum(xa / den + b_ref[...], 0.0)
            h = jnp.dot(x, w_ref[...], preferred_element_type=F32)
            h_s[...] = h
            es_ref[...] = jnp.dot(h, as_ref[...], preferred_element_type=F32)
            ed_ref[...] = jnp.dot(h, ad_ref[...], preferred_element_type=F32)

        for cc in range(ncha):
            @pl.when(c == cc)
            def _():
                lo, hi, off = _chunk_cols(dout, cc)
                pieces = []
                if cc == 0:
                    pieces.append(jnp.ones((BN, 16), F32))
                pieces.append(h_s[:, lo:hi])
                pad = CW - off - (hi - lo)
                if pad:
                    pieces.append(jnp.zeros((BN, pad), F32))
                h3_ref[...] = jnp.concatenate(pieces, 1).reshape(1, BN, CW)

    w_spec = pl.BlockSpec((din, dout), lambda i, c: (0, 0))
    a_spec = pl.BlockSpec((dout, 1), lambda i, c: (0, 0))
    if first:
        in_specs = [pl.BlockSpec((BN, din), lambda i, c: (i, 0)),
                    w_spec, a_spec, a_spec]
    else:
        in_specs = [pl.BlockSpec((2, nchap, BN, CW), lambda i, c: (0, 0, i, 0)),
                    pl.BlockSpec((1, dprev), lambda i, c: (0, 0)),
                    w_spec, a_spec, a_spec]
    out_specs = [pl.BlockSpec((1, BN, CW), lambda i, c: (c, i, 0)),
                 pl.BlockSpec((BN, 1), lambda i, c: (i, 0)),
                 pl.BlockSpec((BN, 1), lambda i, c: (i, 0))]
    out_shape = [jax.ShapeDtypeStruct((ncha, N, CW), F32),
                 jax.ShapeDtypeStruct((N, 1), F32),
                 jax.ShapeDtypeStruct((N, 1), F32)]
    return pl.pallas_call(
        body, grid=(NI, ncha), in_specs=in_specs, out_specs=out_specs,
        out_shape=out_shape,
        scratch_shapes=[pltpu.VMEM((BN, dout), F32)],
        compiler_params=pltpu.CompilerParams(
            dimension_semantics=("arbitrary", "arbitrary")))


def _smax():
    def body(es_ref, s_ref):
        s_ref[...] = jnp.full((1, 16), jnp.max(es_ref[...]), F32)

    return pl.pallas_call(
        body,
        in_specs=[pl.BlockSpec((N, 1), lambda: (0, 0))],
        out_specs=pl.BlockSpec((1, 16), lambda: (0, 0)),
        out_shape=jax.ShapeDtypeStruct((1, 16), F32))


def _epilogue():
    def body(p_ref, b_ref, o_ref):
        pr = p_ref[...]
        p = pr[0, 0] + pr[1, 0]
        den = p[:, 0:1] + 1e-16
        o_ref[...] = p[:, 16:32] / den + b_ref[...]

    return pl.pallas_call(
        body, grid=(NI,),
        in_specs=[pl.BlockSpec((2, 1, BN, CW), lambda i: (0, 0, i, 0)),
                  pl.BlockSpec((1, 16), lambda i: (0, 0))],
        out_specs=pl.BlockSpec((BN, 16), lambda i: (i, 0)),
        out_shape=jax.ShapeDtypeStruct((N, 16), F32))


def _make_sc(layer):
    ncha = _NCHA[layer]
    dout = _DIMS[layer]
    mesh = plsc.VectorSubcoreMesh(core_axis_name="core",
                                  subcore_axis_name="subcore")
    kcs = []
    for cc in range(ncha):
        lo, hi, off = _chunk_cols(dout, cc)
        used = off + (hi - lo)
        kcs.append(-(-used // 16))

    @functools.partial(
        pl.kernel, mesh=mesh,
        compiler_params=pltpu.CompilerParams(needs_layout_passes=False),
        out_type=jax.ShapeDtypeStruct((NCORE, ncha, N, CW), F32),
        scratch_types=[
            pltpu.VMEM((N,), F32),
            pltpu.VMEM((N,), F32),
            pltpu.VMEM((16,), F32),
            pltpu.VMEM((2, B), jnp.int32),
            pltpu.VMEM((2, B), jnp.int32),
            pltpu.VMEM((2, B), jnp.int32),
            pltpu.VMEM((2, B), jnp.int32),
            pltpu.VMEM((B, CW), F32),
            pltpu.VMEM((B, CW), F32),
            pltpu.VMEM_SHARED((N, CW), F32),
            pltpu.SemaphoreType.DMA,
            pltpu.SemaphoreType.DMA,
            pltpu.SemaphoreType.DMA,
            pltpu.SemaphoreType.DMA,
            pltpu.SemaphoreType.DMA,
            pltpu.SemaphoreType.DMA,
            pltpu.SemaphoreType.DMA,
            pltpu.SemaphoreType.DMA,
            pltpu.SemaphoreType.DMA,
        ])
    def k(h3_hbm, es_hbm, ed_hbm, smax_hbm, sd_hbm, out_hbm,
          es_v, ed_v, smax_v, sd0, sd1, sd2, sd3, rows0, rows1, acc_sh,
          sem_g0, sem_g1, sem_s0, sem_s1, sem_b,
          sem_i0, sem_i1, sem_i2, sem_i3):
        cid = lax.axis_index("core")
        sid = lax.axis_index("subcore")
        wid = cid * NSUB + sid
        pltpu.sync_copy(es_hbm, es_v)
        pltpu.sync_copy(ed_hbm, ed_v)
        pltpu.sync_copy(smax_hbm, smax_v)
        sdb = (sd0, sd1, sd2, sd3)
        rows = (rows0, rows1)
        sem_g = (sem_g0, sem_g1)
        sem_s = (sem_s0, sem_s1)
        sem_i = (sem_i0, sem_i1, sem_i2, sem_i3)
        base = wid * NB

        NU = N // 16
        NK = (NU + NSUB - 1) // NSUB
        for c in range(ncha):
            kc = kcs[c]

            @pl.loop(0, 16)
            def _(r):
                for kk in range(KC):
                    rows0[r, pl.ds(16 * kk, 16)] = jnp.zeros((16,), F32)

            @pl.loop(0, NK)
            def _(k):
                u = sid + NSUB * k

                @pl.when(u < NU)
                def _():
                    off = pl.multiple_of(16 * u, 16)
                    pltpu.async_copy(rows0.at[pl.ds(0, 16)],
                                     acc_sh.at[pl.ds(off, 16)], sem_b)

            @pl.loop(0, NK)
            def _(k):
                u = sid + NSUB * k

                @pl.when(u < NU)
                def _():
                    off = pl.multiple_of(16 * u, 16)
                    pltpu.make_async_copy(rows0.at[pl.ds(0, 16)],
                                          acc_sh.at[pl.ds(off, 16)],
                                          sem_b).wait()

            plsc.subcore_barrier()

            for m in range(3):
                pltpu.async_copy(sd_hbm.at[base + m], sdb[m], sem_i[m])
            pltpu.make_async_copy(sd_hbm.at[base], sd0, sem_i0).wait()
            pltpu.async_copy(h3_hbm.at[c].at[sd0.at[0]], rows0, sem_g0)

            @pl.loop(0, (NB + 3) // 4)
            def _(i):
                for bb in range(4):
                    jb = 4 * i + bb
                    p, q = bb % 2, 1 - bb % 2
                    r1, r3 = (bb + 1) % 4, (bb + 3) % 4

                    @pl.when(jb < NB)
                    def _():
                        @pl.when(jb + 1 < NB)
                        def _():
                            @pl.when(jb >= 1)
                            def _():
                                pltpu.make_async_copy(
                                    rows[q], acc_sh.at[sdb[r3].at[1]],
                                    sem_s[q]).wait()

                            @pl.when(jb + 3 < NB)
                            def _():
                                pltpu.async_copy(sd_hbm.at[base + jb + 3],
                                                 sdb[r3], sem_i[r3])
                            pltpu.make_async_copy(sd_hbm.at[base + jb + 1],
                                                  sdb[r1], sem_i[r1]).wait()
                            pltpu.async_copy(h3_hbm.at[c].at[sdb[r1].at[0]],
                                             rows[q], sem_g[q])

                        pltpu.make_async_copy(h3_hbm.at[c].at[sdb[bb].at[0]],
                                              rows[p], sem_g[p]).wait()
                        sv = smax_v[...]

                        @pl.loop(0, B // 16)
                        def _(qq):
                            s16 = sdb[bb][0, pl.ds(16 * qq, 16)]
                            d16 = sdb[bb][1, pl.ds(16 * qq, 16)]
                            esg = plsc.load_gather(es_v, [s16])
                            edg = plsc.load_gather(ed_v, [d16])
                            uu = esg + edg
                            e = jnp.where(uu >= 0, uu, 0.2 * uu)
                            vv = edg + sv
                            mt = jnp.where(vv >= 0, vv, 0.2 * vv)
                            ex16 = jnp.exp(e - mt)
                            for t in range(16):
                                s = ex16[t]
                                for kk in range(kc):
                                    rows[p][16 * qq + t, pl.ds(16 * kk, 16)] = (
                                        rows[p][16 * qq + t,
                                                pl.ds(16 * kk, 16)] * s)

                        pltpu.async_copy(rows[p], acc_sh.at[sdb[bb].at[1]],
                                         sem_s[p], add=True)

            pltpu.make_async_copy(rows[(NB - 1) % 2],
                                  acc_sh.at[sdb[(NB - 1) % 4].at[1]],
                                  sem_s[(NB - 1) % 2]).wait()
            pltpu.make_async_copy(rows[(NB - 2) % 2],
                                  acc_sh.at[sdb[(NB - 2) % 4].at[1]],
                                  sem_s[(NB - 2) % 2]).wait()
            plsc.subcore_barrier()

            @pl.loop(0, NK)
            def _(k):
                u = sid + NSUB * k

                @pl.when(u < NU)
                def _():
                    off = pl.multiple_of(16 * u, 16)
                    pltpu.async_copy(acc_sh.at[pl.ds(off, 16)],
                                     out_hbm.at[cid, c, pl.ds(off, 16)],
                                     sem_b)

            @pl.loop(0, NK)
            def _(k):
                u = sid + NSUB * k

                @pl.when(u < NU)
                def _():
                    off = pl.multiple_of(16 * u, 16)
                    pltpu.make_async_copy(acc_sh.at[pl.ds(off, 16)],
                                          out_hbm.at[cid, c, pl.ds(off, 16)],
                                          sem_b).wait()

    return k


_TC = {i: _make_tc(i) for i in range(1, 7)}
_SC = {i: _make_sc(i) for i in range(1, 7)}
_EPI = _epilogue()
_SMAX = _smax()


def kernel(points, edge_index, W1, a_src1, a_dst1, b1, W2, a_src2, a_dst2, b2,
           W3, a_src3, a_dst3, b3, W4, a_src4, a_dst4, b4,
           W5, a_src5, a_dst5, b5, W6, a_src6, a_dst6, b6):
    Ws = [W1, W2, W3, W4, W5, W6]
    ass = [a_src1, a_src2, a_src3, a_src4, a_src5, a_src6]
    ads = [a_dst1, a_dst2, a_dst3, a_dst4, a_dst5, a_dst6]
    bs = [b1, b2, b3, b4, b5, b6]
    x = points.reshape(N, _DIMS[0])
    sdr = jnp.concatenate([edge_index[0].reshape(NW * NB, 1, B),
                           edge_index[1].reshape(NW * NB, 1, B)], 1)
    parts = None
    for i in range(1, 7):
        W = Ws[i - 1]
        a_s = ass[i - 1].reshape(-1, 1)
        a_d = ads[i - 1].reshape(-1, 1)
        if i == 1:
            h3, es, ed = _TC[1](x, W, a_s, a_d)
        else:
            bprev = bs[i - 2].reshape(1, -1)
            h3, es, ed = _TC[i](parts, bprev, W, a_s, a_d)
        smax = _SMAX(es)
        parts = _SC[i](h3, es.reshape(N), ed.reshape(N), smax.reshape(16),
                       sdr)
    b6p = jnp.zeros((1, 16), F32).at[0, :3].set(b6)
    out16 = _EPI(parts, b6p)
    return out16[:, :3].reshape(1, N, 3)


# speedup vs baseline: 25.4488x; 1.0538x over previous
import functools

import jax
import jax.numpy as jnp
from jax import lax
from jax.experimental import pallas as pl
from jax.experimental.pallas import tpu as pltpu
from jax.experimental.pallas import tpu_sc as plsc

N = 10000
E = 320000
NCORE = 2
NSUB = 16
NW = NCORE * NSUB
EPT = E // NW
B = 80
NB = EPT // B
BN = 1000
NI = N // BN
CW = 128
KC = CW // 16
F32 = jnp.float32

_DIMS = [128, 64, 256, 1024, 256, 64, 3]
_ONES = {1: True, 2: False, 3: False, 4: False, 5: True, 6: True}
_NCHF = {i: 1 if _ONES[i] else _DIMS[i] // CW for i in range(1, 7)}
_NCHO = {i: _NCHF[i] + (0 if _ONES[i] else 1) for i in range(1, 7)}


def _chunk_cols(dout, cc, ones):
    if ones:
        if cc == 0:
            return 0, min(dout, CW - 16), 16
        lo = (CW - 16) + (cc - 1) * CW
        return lo, min(dout, lo + CW), 0
    lo = cc * CW
    return lo, min(dout, lo + CW), 0


def _make_tc(layer):
    ncha = _NCHF[layer]
    ones = _ONES[layer]
    dout = _DIMS[layer]
    din = _DIMS[layer - 1]
    first = layer == 1
    if not first:
        dprev, nchop = din, _NCHO[layer - 1]
        nchfp, onesp = _NCHF[layer - 1], _ONES[layer - 1]

    def body(*refs):
        if first:
            x_ref, w_ref, as_ref, ad_ref, h3_ref, es_ref, ed_ref, h_s = refs
        else:
            (p_ref, b_ref, w_ref, as_ref, ad_ref,
             h3_ref, es_ref, ed_ref, h_s) = refs
        c = pl.program_id(1)

        @pl.when(c == 0)
        def _():
            if first:
                x = x_ref[...]
            else:
                pr = p_ref[...]
                p = pr[0] + pr[1]
                dcc = 0 if onesp else nchfp
                den = p[dcc, :, 0:1] + 1e-16
                cols = []
                for cc in range(nchfp):
                    lo, hi, off = _chunk_cols(dprev, cc, onesp)
                    cols.append(p[cc, :, off:off + (hi - lo)])
                xa = cols[0] if len(cols) == 1 else jnp.concatenate(cols, 1)
                x = jnp.maximum(xa / den + b_ref[...], 0.0)
            h = jnp.dot(x, w_ref[...], preferred_element_type=F32)
            h_s[...] = h
            es_ref[...] = jnp.dot(h, as_ref[...], preferred_element_type=F32)
            ed_ref[...] = jnp.dot(h, ad_ref[...], preferred_element_type=F32)

        for cc in range(ncha):
            @pl.when(c == cc)
            def _():
                lo, hi, off = _chunk_cols(dout, cc, ones)
                pieces = []
                if ones and cc == 0:
                    pieces.append(jnp.ones((BN, 16), F32))
                pieces.append(h_s[:, lo:hi])
                pad = CW - off - (hi - lo)
                if pad:
                    pieces.append(jnp.zeros((BN, pad), F32))
                h3_ref[...] = (pieces[0] if len(pieces) == 1 else
                               jnp.concatenate(pieces, 1)).reshape(1, BN, CW)

    w_spec = pl.BlockSpec((din, dout), lambda i, c: (0, 0))
    a_spec = pl.BlockSpec((dout, 1), lambda i, c: (0, 0))
    if first:
        in_specs = [pl.BlockSpec((BN, din), lambda i, c: (i, 0)),
                    w_spec, a_spec, a_spec]
    else:
        in_specs = [pl.BlockSpec((2, nchop, BN, CW), lambda i, c: (0, 0, i, 0)),
                    pl.BlockSpec((1, dprev), lambda i, c: (0, 0)),
                    w_spec, a_spec, a_spec]
    out_specs = [pl.BlockSpec((1, BN, CW), lambda i, c: (c, i, 0)),
                 pl.BlockSpec((BN, 1), lambda i, c: (i, 0)),
                 pl.BlockSpec((BN, 1), lambda i, c: (i, 0))]
    out_shape = [jax.ShapeDtypeStruct((ncha, N, CW), F32),
                 jax.ShapeDtypeStruct((N, 1), F32),
                 jax.ShapeDtypeStruct((N, 1), F32)]
    return pl.pallas_call(
        body, grid=(NI, ncha), in_specs=in_specs, out_specs=out_specs,
        out_shape=out_shape,
        scratch_shapes=[pltpu.VMEM((BN, dout), F32)],
        compiler_params=pltpu.CompilerParams(
            dimension_semantics=("arbitrary", "arbitrary")))


def _smax():
    def body(es_ref, s_ref):
        s_ref[...] = jnp.full((1, 16), jnp.max(es_ref[...]), F32)

    return pl.pallas_call(
        body,
        in_specs=[pl.BlockSpec((N, 1), lambda: (0, 0))],
        out_specs=pl.BlockSpec((1, 16), lambda: (0, 0)),
        out_shape=jax.ShapeDtypeStruct((1, 16), F32))


def _epilogue():
    def body(p_ref, b_ref, o_ref):
        pr = p_ref[...]
        p = pr[0, 0] + pr[1, 0]
        den = p[:, 0:1] + 1e-16
        o_ref[...] = p[:, 16:32] / den + b_ref[...]

    return pl.pallas_call(
        body, grid=(NI,),
        in_specs=[pl.BlockSpec((2, 1, BN, CW), lambda i: (0, 0, i, 0)),
                  pl.BlockSpec((1, 16), lambda i: (0, 0))],
        out_specs=pl.BlockSpec((BN, 16), lambda i: (i, 0)),
        out_shape=jax.ShapeDtypeStruct((N, 16), F32))


def _make_sc(layer):
    ncha = _NCHF[layer]
    ones = _ONES[layer]
    ncho = _NCHO[layer]
    dout = _DIMS[layer]
    mesh = plsc.VectorSubcoreMesh(core_axis_name="core",
                                  subcore_axis_name="subcore")
    kcs = []
    for cc in range(ncha):
        lo, hi, off = _chunk_cols(dout, cc, ones)
        used = off + (hi - lo)
        kcs.append(-(-used // 16))

    @functools.partial(
        pl.kernel, mesh=mesh,
        compiler_params=pltpu.CompilerParams(needs_layout_passes=False),
        out_type=jax.ShapeDtypeStruct((NCORE, ncho, N, CW), F32),
        scratch_types=[
            pltpu.VMEM((N,), F32),
            pltpu.VMEM((N,), F32),
            pltpu.VMEM((16,), F32),
            pltpu.VMEM((2, B), jnp.int32),
            pltpu.VMEM((2, B), jnp.int32),
            pltpu.VMEM((2, B), jnp.int32),
            pltpu.VMEM((2, B), jnp.int32),
            pltpu.VMEM((B, CW), F32),
            pltpu.VMEM((B, CW), F32),
            pltpu.VMEM_SHARED((N, CW), F32),
            pltpu.SemaphoreType.DMA,
            pltpu.SemaphoreType.DMA,
            pltpu.SemaphoreType.DMA,
            pltpu.SemaphoreType.DMA,
            pltpu.SemaphoreType.DMA,
            pltpu.SemaphoreType.DMA,
            pltpu.SemaphoreType.DMA,
            pltpu.SemaphoreType.DMA,
            pltpu.SemaphoreType.DMA,
        ])
    def k(h3_hbm, es_hbm, ed_hbm, smax_hbm, sd_hbm, out_hbm,
          es_v, ed_v, smax_v, sd0, sd1, sd2, sd3, rows0, rows1, acc_sh,
          sem_g0, sem_g1, sem_s0, sem_s1, sem_b,
          sem_i0, sem_i1, sem_i2, sem_i3):
        cid = lax.axis_index("core")
        sid = lax.axis_index("subcore")
        wid = cid * NSUB + sid
        pltpu.sync_copy(es_hbm, es_v)
        pltpu.sync_copy(ed_hbm, ed_v)
        pltpu.sync_copy(smax_hbm, smax_v)
        sdb = (sd0, sd1, sd2, sd3)
        rows = (rows0, rows1)
        sem_g = (sem_g0, sem_g1)
        sem_s = (sem_s0, sem_s1)
        sem_i = (sem_i0, sem_i1, sem_i2, sem_i3)
        base = wid * NB

        NU = N // 16
        NK = (NU + NSUB - 1) // NSUB
        for c in range(ncho):
            feat = c < ncha
            kc = kcs[c] if feat else 1

            @pl.loop(0, 16)
            def _(r):
                for kk in range(KC):
                    rows0[r, pl.ds(16 * kk, 16)] = jnp.zeros((16,), F32)

            @pl.loop(0, NK)
            def _(k):
                u = sid + NSUB * k

                @pl.when(u < NU)
                def _():
                    off = pl.multiple_of(16 * u, 16)
                    pltpu.async_copy(rows0.at[pl.ds(0, 16)],
                                     acc_sh.at[pl.ds(off, 16)], sem_b)

            @pl.loop(0, NK)
            def _(k):
                u = sid + NSUB * k

                @pl.when(u < NU)
                def _():
                    off = pl.multiple_of(16 * u, 16)
                    pltpu.make_async_copy(rows0.at[pl.ds(0, 16)],
                                          acc_sh.at[pl.ds(off, 16)],
                                          sem_b).wait()

            plsc.subcore_barrier()

            for m in range(3):
                pltpu.async_copy(sd_hbm.at[base + m], sdb[m], sem_i[m])
            if feat:
                pltpu.make_async_copy(sd_hbm.at[base], sd0, sem_i0).wait()
                pltpu.async_copy(h3_hbm.at[c].at[sd0.at[0]], rows0, sem_g0)

            @pl.loop(0, (NB + 3) // 4)
            def _(i):
                for bb in range(4):
                    jb = 4 * i + bb
                    p, q = bb % 2, 1 - bb % 2
                    r1, r3 = (bb + 1) % 4, (bb + 3) % 4

                    @pl.when(jb < NB)
                    def _():
                        @pl.when(jb + 1 < NB)
                        def _():
                            @pl.when(jb >= 1)
                            def _():
                                pltpu.make_async_copy(
                                    rows[q], acc_sh.at[sdb[r3].at[1]],
                                    sem_s[q]).wait()

                            @pl.when(jb + 3 < NB)
                            def _():
                                pltpu.async_copy(sd_hbm.at[base + jb + 3],
                                                 sdb[r3], sem_i[r3])
                            if feat:
                                pltpu.make_async_copy(sd_hbm.at[base + jb + 1],
                                                      sdb[r1],
                                                      sem_i[r1]).wait()
                                pltpu.async_copy(h3_hbm.at[c].at[sdb[r1].at[0]],
                                                 rows[q], sem_g[q])

                        if feat:
                            pltpu.make_async_copy(
                                h3_hbm.at[c].at[sdb[bb].at[0]],
                                rows[p], sem_g[p]).wait()
                        else:
                            pltpu.make_async_copy(sd_hbm.at[base + jb],
                                                  sdb[bb], sem_i[bb]).wait()
                        sv = smax_v[...]

                        @pl.loop(0, B // 16)
                        def _(qq):
                            s16 = sdb[bb][0, pl.ds(16 * qq, 16)]
                            d16 = sdb[bb][1, pl.ds(16 * qq, 16)]
                            esg = plsc.load_gather(es_v, [s16])
                            edg = plsc.load_gather(ed_v, [d16])
                            uu = esg + edg
                            e = jnp.where(uu >= 0, uu, 0.2 * uu)
                            vv = edg + sv
                            mt = jnp.where(vv >= 0, vv, 0.2 * vv)
                            ex16 = jnp.exp(e - mt)
                            if feat:
                                for t in range(16):
                                    s = ex16[t]
                                    for kk in range(kc):
                                        rows[p][16 * qq + t,
                                                pl.ds(16 * kk, 16)] = (
                                            rows[p][16 * qq + t,
                                                    pl.ds(16 * kk, 16)] * s)
                            else:
                                li = lax.iota(jnp.int32, 16)
                                for t in range(16):
                                    rows[p][16 * qq + t, pl.ds(0, 16)] = (
                                        jnp.where(li == 0, ex16[t], 0.0))

                        pltpu.async_copy(rows[p], acc_sh.at[sdb[bb].at[1]],
                                         sem_s[p], add=True)

            pltpu.make_async_copy(rows[(NB - 1) % 2],
                                  acc_sh.at[sdb[(NB - 1) % 4].at[1]],
                                  sem_s[(NB - 1) % 2]).wait()
            pltpu.make_async_copy(rows[(NB - 2) % 2],
                                  acc_sh.at[sdb[(NB - 2) % 4].at[1]],
                                  sem_s[(NB - 2) % 2]).wait()
            plsc.subcore_barrier()

            @pl.loop(0, NK)
            def _(k):
                u = sid + NSUB * k

                @pl.when(u < NU)
                def _():
                    off = pl.multiple_of(16 * u, 16)
                    pltpu.async_copy(acc_sh.at[pl.ds(off, 16)],
                                     out_hbm.at[cid, c, pl.ds(off, 16)],
                                     sem_b)

            @pl.loop(0, NK)
            def _(k):
                u = sid + NSUB * k

                @pl.when(u < NU)
                def _():
                    off = pl.multiple_of(16 * u, 16)
                    pltpu.make_async_copy(acc_sh.at[pl.ds(off, 16)],
                                          out_hbm.at[cid, c, pl.ds(off, 16)],
                                          sem_b).wait()

    return k


_TC = {i: _make_tc(i) for i in range(1, 7)}
_SC = {i: _make_sc(i) for i in range(1, 7)}
_EPI = _epilogue()
_SMAX = _smax()


def kernel(points, edge_index, W1, a_src1, a_dst1, b1, W2, a_src2, a_dst2, b2,
           W3, a_src3, a_dst3, b3, W4, a_src4, a_dst4, b4,
           W5, a_src5, a_dst5, b5, W6, a_src6, a_dst6, b6):
    Ws = [W1, W2, W3, W4, W5, W6]
    ass = [a_src1, a_src2, a_src3, a_src4, a_src5, a_src6]
    ads = [a_dst1, a_dst2, a_dst3, a_dst4, a_dst5, a_dst6]
    bs = [b1, b2, b3, b4, b5, b6]
    x = points.reshape(N, _DIMS[0])
    sdr = jnp.concatenate([edge_index[0].reshape(NW * NB, 1, B),
                           edge_index[1].reshape(NW * NB, 1, B)], 1)
    parts = None
    for i in range(1, 7):
        W = Ws[i - 1]
        a_s = ass[i - 1].reshape(-1, 1)
        a_d = ads[i - 1].reshape(-1, 1)
        if i == 1:
            h3, es, ed = _TC[1](x, W, a_s, a_d)
        else:
            bprev = bs[i - 2].reshape(1, -1)
            h3, es, ed = _TC[i](parts, bprev, W, a_s, a_d)
        smax = _SMAX(es)
        parts = _SC[i](h3, es.reshape(N), ed.reshape(N), smax.reshape(16),
                       sdr)
    b6p = jnp.zeros((1, 16), F32).at[0, :3].set(b6)
    out16 = _EPI(parts, b6p)
    return out16[:, :3].reshape(1, N, 3)
